# baseline (speedup 1.0000x reference)
import jax
import jax.numpy as jnp
from jax import lax
from jax.experimental import pallas as pl
from jax.experimental.pallas import tpu as pltpu

B = 1024
T = 200
D = 128
_NB = 4
_RB = B // _NB
_UNROLL = 8


def _eos_gather_body(x_hbm, text_hbm, out_hbm, text_v, flat_v, flat_s, out_v,
                     st0, st1, st2, st3, sg0, sg1, sg2, sg3, sem_s, sem_o):
    st = (st0, st1, st2, st3)
    sg = (sg0, sg1, sg2, sg3)

    def stage_copy(b):
        return pltpu.make_async_copy(
            text_hbm.at[pl.ds(b * _RB, _RB)],
            text_v.at[pl.ds(b * _RB, _RB)],
            st[b],
        )

    for b in range(_NB):
        stage_copy(b).start(priority=b % 2)

    for b in range(_NB):
        stage_copy(b).wait()
        t = text_v[pl.ds(b * _RB, _RB), :]
        cnt = jnp.sum((t != 0).astype(jnp.int32), axis=1)
        eos = jnp.clip(cnt - 1, 0, T - 1)
        base = (lax.broadcasted_iota(jnp.int32, (_RB,), 0) + b * _RB) * T
        flat_v[pl.ds(b * _RB, _RB)] = base + eos

        smem_copy = pltpu.make_async_copy(
            flat_v.at[pl.ds(b * _RB, _RB)],
            flat_s.at[pl.ds(b * _RB, _RB)],
            sem_s,
        )
        smem_copy.start()
        smem_copy.wait()

        def issue(i, carry, b=b):
            for u in range(_UNROLL):
                ii = b * _RB + i * _UNROLL + u
                r = flat_s[ii]
                pltpu.make_async_copy(
                    x_hbm.at[pl.ds(r, 1)], out_v.at[pl.ds(ii, 1)], sg[b]
                ).start(priority=u % 2)
            return carry

        lax.fori_loop(0, _RB // _UNROLL, issue, 0)

    def out_copy(b):
        return pltpu.make_async_copy(
            out_v.at[pl.ds(b * _RB, _RB)],
            out_hbm.at[pl.ds(b * _RB, _RB)],
            sem_o,
        )

    for b in range(_NB):
        pltpu.make_async_copy(
            x_hbm.at[pl.ds(0, _RB)], out_v.at[pl.ds(b * _RB, _RB)], sg[b]
        ).wait()
        out_copy(b).start(priority=b % 2)
    for b in range(_NB):
        out_copy(b).wait()


@jax.jit
def kernel(x, text):
    x2 = x.reshape(B * T, D)
    text32 = text.astype(jnp.int32)
    return pl.pallas_call(
        _eos_gather_body,
        in_specs=[
            pl.BlockSpec(memory_space=pl.ANY),
            pl.BlockSpec(memory_space=pl.ANY),
        ],
        out_specs=pl.BlockSpec(memory_space=pl.ANY),
        out_shape=jax.ShapeDtypeStruct((B, D), jnp.float32),
        scratch_shapes=[
            pltpu.VMEM((B, T), jnp.int32),
            pltpu.VMEM((B,), jnp.int32),
            pltpu.SMEM((B,), jnp.int32),
            pltpu.VMEM((B, D), jnp.float32),
        ] + [pltpu.SemaphoreType.DMA] * 10,
    )(x2, text32)


# speedup vs baseline: 1.1759x; 1.1759x over previous
import jax
import jax.numpy as jnp
from jax import lax
from jax.experimental import pallas as pl
from jax.experimental.pallas import tpu as pltpu

B = 1024
T = 200
D = 128
_H = B // 2
_UNROLL = 8


def _eos_gather_body(x_hbm, text_ref, out_ref, flat_v, flat_s, sem0, sem1):
    def count_half(h):
        t = text_ref[pl.ds(h * _H, _H), :]
        cnt = jnp.sum((t != 0).astype(jnp.int32), axis=1)
        eos = jnp.clip(cnt - 1, 0, T - 1)
        base = (lax.broadcasted_iota(jnp.int32, (_H,), 0) + h * _H) * T
        flat_v[pl.ds(h * _H, _H)] = base + eos

    def smem_copy(h):
        return pltpu.make_async_copy(
            flat_v.at[pl.ds(h * _H, _H)], flat_s.at[pl.ds(h * _H, _H)], sem0
        )

    def issue_half(h):
        def issue(i, carry):
            for u in range(_UNROLL):
                ii = h * _H + i * _UNROLL + u
                r = flat_s[ii]
                pltpu.make_async_copy(
                    x_hbm.at[pl.ds(r, 1)], out_ref.at[pl.ds(ii, 1)], sem1
                ).start(priority=u % 2)
            return carry

        lax.fori_loop(0, _H // _UNROLL, issue, 0)

    count_half(0)
    smem_copy(0).start()
    count_half(1)
    smem_copy(1).start()
    smem_copy(0).wait()
    issue_half(0)
    smem_copy(1).wait()
    issue_half(1)

    pltpu.make_async_copy(x_hbm.at[pl.ds(0, B)], out_ref, sem1).wait()


@jax.jit
def kernel(x, text):
    x2 = x.reshape(B * T, D)
    text32 = text.astype(jnp.int32)
    return pl.pallas_call(
        _eos_gather_body,
        in_specs=[
            pl.BlockSpec(memory_space=pl.ANY),
            pl.BlockSpec(memory_space=pltpu.VMEM),
        ],
        out_specs=pl.BlockSpec(memory_space=pltpu.VMEM),
        out_shape=jax.ShapeDtypeStruct((B, D), jnp.float32),
        scratch_shapes=[
            pltpu.VMEM((B,), jnp.int32),
            pltpu.SMEM((B,), jnp.int32),
            pltpu.SemaphoreType.DMA,
            pltpu.SemaphoreType.DMA,
        ],
    )(x2, text32)
